# baseline (speedup 1.0000x reference)
import jax
import jax.numpy as jnp
from jax import lax
from jax.experimental import pallas as pl
from jax.experimental.pallas import tpu as pltpu
from jax.experimental.pallas import tpu_sc as plsc

N_CLS = 1000
DIM = 768
N_CTX = 4
SUF = 72
ROWS = 1 + N_CTX + SUF
SHIFT = 1 + N_CTX
LANES = 16
NCOL = DIM // LANES

NW = 32
BLK = 32


def _copy_row(dst, dr, src, sr):
    vals = [src[sr, pl.ds(cc * LANES, LANES)] for cc in range(NCOL)]
    for cc in range(NCOL):
        dst[dr, pl.ds(cc * LANES, LANES)] = vals[cc]


def _sc_body(
    prefix_hbm, ctx_hbm, suffix_hbm, out_hbm,
    bufO0, bufO1, bufP0, bufP1, bufC, f0, f1, s0, s1,
):
    c = lax.axis_index("c")
    s = lax.axis_index("s")
    wid = s * 2 + c
    base = jnp.minimum(wid * BLK, N_CLS - BLK)

    bufO = (bufO0, bufO1)
    bufP = (bufP0, bufP1)
    fsem = (f0, f1)
    ssem = (s0, s1)

    pltpu.sync_copy(ctx_hbm, bufC)

    def gathers(j, b):
        i = base + j
        return (
            pltpu.make_async_copy(
                suffix_hbm.at[i], bufO[b].at[pl.ds(0, SUF)], fsem[b]
            ),
            pltpu.make_async_copy(prefix_hbm.at[i], bufP[b], fsem[b]),
        )

    def store(j, b):
        return pltpu.make_async_copy(bufO[b], out_hbm.at[base + j], ssem[b])

    def assemble(b):
        @plsc.parallel_loop(0, SUF, unroll=4)
        def sh(k):
            r = ROWS - 1 - k
            _copy_row(bufO[b], r, bufO[b], r - SHIFT)
        _copy_row(bufO[b], 0, bufP[b], 0)
        for r in range(N_CTX):
            _copy_row(bufO[b], 1 + r, bufC, r)

    def fire(cps):
        for cp in cps:
            cp.start()

    def drain(cps):
        for cp in cps:
            cp.wait()

    fire(gathers(0, 0))

    def body(k, _):
        j0 = 2 * k
        j1 = j0 + 1
        drain(gathers(j0, 0))

        @pl.when(k >= 1)
        def _():
            drain((store(j0 - 1, 1),))

        fire(gathers(j1, 1))
        assemble(0)
        fire((store(j0, 0),))
        drain(gathers(j1, 1))
        drain((store(j0, 0),))

        @pl.when(k < BLK // 2 - 1)
        def _():
            fire(gathers(j0 + 2, 0))

        assemble(1)
        fire((store(j1, 1),))
        return 0

    lax.fori_loop(0, BLK // 2, body, 0)
    drain((store(BLK - 1, 1),))


def kernel(token_prefix, ctx, token_suffix):
    return pl.kernel(
        _sc_body,
        out_type=jax.ShapeDtypeStruct((N_CLS, ROWS, DIM), jnp.float32),
        mesh=plsc.VectorSubcoreMesh(core_axis_name="c", subcore_axis_name="s"),
        scratch_types=[
            pltpu.VMEM((ROWS, DIM), jnp.float32),
            pltpu.VMEM((ROWS, DIM), jnp.float32),
            pltpu.VMEM((1, DIM), jnp.float32),
            pltpu.VMEM((1, DIM), jnp.float32),
            pltpu.VMEM((N_CTX, DIM), jnp.float32),
            pltpu.SemaphoreType.DMA,
            pltpu.SemaphoreType.DMA,
            pltpu.SemaphoreType.DMA,
            pltpu.SemaphoreType.DMA,
        ],
    )(token_prefix, ctx, token_suffix)


# speedup vs baseline: 1.0106x; 1.0106x over previous
import jax
import jax.numpy as jnp
from jax import lax
from jax.experimental import pallas as pl
from jax.experimental.pallas import tpu as pltpu
from jax.experimental.pallas import tpu_sc as plsc

N_CLS = 1000
DIM = 768
N_CTX = 4
SUF = 72
ROWS = 1 + N_CTX + SUF
SHIFT = 1 + N_CTX
LANES = 16
HALF = DIM // 2
NCOL = HALF // LANES

NW = 32
BLK = 32
NBUF = 4
NITEMS = 2 * BLK


def _copy_row(dst, dr, doff, src, sr, soff):
    vals = [src[sr, pl.ds(soff + cc * LANES, LANES)] for cc in range(NCOL)]
    for cc in range(NCOL):
        dst[dr, pl.ds(doff + cc * LANES, LANES)] = vals[cc]


def _sc_body(
    prefix_hbm, ctx_hbm, suffix_hbm, out_hbm,
    b0, b1, b2, b3, p0, p1, bufC,
    f0, f1, f2, f3, s0, s1, s2, s3,
):
    c = lax.axis_index("c")
    s = lax.axis_index("s")
    wid = s * 2 + c
    base = jnp.minimum(wid * BLK, N_CLS - BLK)

    bufs = (b0, b1, b2, b3)
    bufP = (p0, p1)
    fsem = (f0, f1, f2, f3)
    ssem = (s0, s1, s2, s3)

    pltpu.sync_copy(ctx_hbm, bufC)

    def gathers(k, p):
        i = base + 2 * k + p // 2
        h = p % 2
        cps = [
            pltpu.make_async_copy(
                suffix_hbm.at[i, :, pl.ds(h * HALF, HALF)],
                bufs[p].at[pl.ds(0, SUF)],
                fsem[p],
            )
        ]
        if h == 0:
            cps.append(
                pltpu.make_async_copy(prefix_hbm.at[i], bufP[p // 2], fsem[p])
            )
        return cps

    def store(k, p):
        i = base + 2 * k + p // 2
        h = p % 2
        return pltpu.make_async_copy(
            bufs[p], out_hbm.at[i, :, pl.ds(h * HALF, HALF)], ssem[p]
        )

    def assemble(p):
        h = p % 2

        @plsc.parallel_loop(0, SUF, unroll=4)
        def sh(q):
            r = ROWS - 1 - q
            _copy_row(bufs[p], r, 0, bufs[p], r - SHIFT, 0)

        _copy_row(bufs[p], 0, 0, bufP[p // 2], 0, h * HALF)
        for r in range(N_CTX):
            _copy_row(bufs[p], 1 + r, 0, bufC, r, h * HALF)

    def fire(cps):
        for cp in cps:
            cp.start()

    def drain(cps):
        for cp in cps:
            cp.wait()

    fire(gathers(0, 0))
    fire(gathers(0, 1))

    def body(k, _):
        for p in range(NBUF):
            drain(gathers(k, p))
            assemble(p)
            fire((store(k, p),))
            if p < 2:

                @pl.when(k >= 1)
                def _():
                    drain((store(k - 1, p + 2),))

                fire(gathers(k, p + 2))
            else:
                drain((store(k, p - 2),))

                @pl.when(k < BLK // 2 - 1)
                def _():
                    fire(gathers(k + 1, p - 2))

        return 0

    lax.fori_loop(0, BLK // 2, body, 0)
    drain((store(BLK // 2 - 1, 2),))
    drain((store(BLK // 2 - 1, 3),))


def kernel(token_prefix, ctx, token_suffix):
    return pl.kernel(
        _sc_body,
        out_type=jax.ShapeDtypeStruct((N_CLS, ROWS, DIM), jnp.float32),
        mesh=plsc.VectorSubcoreMesh(core_axis_name="c", subcore_axis_name="s"),
        scratch_types=[
            pltpu.VMEM((ROWS, HALF), jnp.float32),
            pltpu.VMEM((ROWS, HALF), jnp.float32),
            pltpu.VMEM((ROWS, HALF), jnp.float32),
            pltpu.VMEM((ROWS, HALF), jnp.float32),
            pltpu.VMEM((1, DIM), jnp.float32),
            pltpu.VMEM((1, DIM), jnp.float32),
            pltpu.VMEM((N_CTX, DIM), jnp.float32),
            pltpu.SemaphoreType.DMA,
            pltpu.SemaphoreType.DMA,
            pltpu.SemaphoreType.DMA,
            pltpu.SemaphoreType.DMA,
            pltpu.SemaphoreType.DMA,
            pltpu.SemaphoreType.DMA,
            pltpu.SemaphoreType.DMA,
            pltpu.SemaphoreType.DMA,
        ],
    )(token_prefix, ctx, token_suffix)
